# baseline (speedup 1.0000x reference)
import functools

import numpy as np
import jax
import jax.numpy as jnp
from jax import lax
from jax.experimental import pallas as pl
from jax.experimental.pallas import tpu as pltpu, tpu_sc as plsc

TOPK = 2
NEXP = 8
T = 2048
D = 768
R = T * TOPK
NT = 16
PER = R // NT
RB = 256
NB = R // RB
G = NB + NEXP - 1


def _router_body(x_ref, wg_ref, idx_ref, sc_ref, lb_ref, rz_ref, cnt_ref):
    xf = x_ref[...]
    logits = jnp.dot(xf, wg_ref[...], preferred_element_type=jnp.float32)
    m = jnp.max(logits, axis=-1, keepdims=True)
    ex = jnp.exp(logits - m)
    ssum = jnp.sum(ex, axis=-1, keepdims=True)
    scores = ex / ssum
    rz = jnp.log(ssum) + m
    rz_ref[0, 0] = jnp.mean(rz * rz)

    col = jax.lax.broadcasted_iota(jnp.int32, scores.shape, 1)
    m1 = jnp.max(scores, axis=-1, keepdims=True)
    idx1 = jnp.min(jnp.where(scores == m1, col, NEXP), axis=-1, keepdims=True)
    sel1 = col == idx1
    s_masked = jnp.where(sel1, -jnp.inf, scores)
    m2 = jnp.max(s_masked, axis=-1, keepdims=True)
    idx2 = jnp.min(jnp.where(s_masked == m2, col, NEXP), axis=-1,
                   keepdims=True)
    sel2 = col == idx2
    picked = sel1 | sel2
    w = jnp.where(picked, scores, 0.0)
    s1 = jnp.max(jnp.where(sel1, scores, 0.0), axis=-1, keepdims=True)
    s2 = jnp.max(jnp.where(sel2, scores, 0.0), axis=-1, keepdims=True)
    idx_ref[...] = jnp.concatenate([idx1, idx2], axis=1)
    sc_ref[...] = jnp.concatenate([s1, s2], axis=1)

    counts = jnp.sum(picked.astype(jnp.int32), axis=0)
    cnt_ref[...] = counts[None, :]
    seg_sum = jnp.sum(w, axis=0)
    total = jnp.float32(T * TOPK)
    dist = counts.astype(jnp.float32) / total
    avg = seg_sum / jnp.maximum(counts.astype(jnp.float32), 1.0)
    lb_ref[0, 0] = jnp.sum(dist * avg) * NEXP


def _router(xf, Wg):
    return pl.pallas_call(
        _router_body,
        in_specs=[pl.BlockSpec((T, D), lambda: (0, 0)),
                  pl.BlockSpec((D, NEXP), lambda: (0, 0))],
        out_specs=[pl.BlockSpec((T, TOPK), lambda: (0, 0)),
                   pl.BlockSpec((T, TOPK), lambda: (0, 0)),
                   pl.BlockSpec(memory_space=pltpu.SMEM),
                   pl.BlockSpec(memory_space=pltpu.SMEM),
                   pl.BlockSpec((1, NEXP), lambda: (0, 0))],
        out_shape=[jax.ShapeDtypeStruct((T, TOPK), jnp.int32),
                   jax.ShapeDtypeStruct((T, TOPK), jnp.float32),
                   jax.ShapeDtypeStruct((1, 1), jnp.float32),
                   jax.ShapeDtypeStruct((1, 1), jnp.float32),
                   jax.ShapeDtypeStruct((1, NEXP), jnp.int32)],
    )(xf, Wg)


def _lanesum(x, lane):
    for sh in (1, 2, 4, 8):
        x = x + x.at[(lane + sh) & 15].get(mode="promise_in_bounds")
    return x


def _prefix_incl(x, lane):
    for sh in (1, 2, 4, 8):
        shifted = x.at[jnp.maximum(lane - sh, 0)].get(mode="promise_in_bounds")
        ge = jnp.minimum(jnp.maximum(lane - (sh - 1), 0), 1)
        x = x + ge * shifted
    return x


def _eqmask(a, b):
    return 1 - jnp.minimum(jnp.abs(a - b), 1)


def _sc_sort_body(eflat, sflat, pos_hbm, rowid_hbm, ssort_hbm,
                  eloc, sloc, hv_v, histcopy, posloc, rbuf, zbuf, zbuf_f,
                  sh_hist, sh_rowid, sh_ssort):
    c = lax.axis_index("c")
    s = lax.axis_index("s")
    lane = lax.iota(jnp.int32, 16)
    zero16 = lane * 0

    @pl.when(c == 0)
    def _work():
        base = s * PER
        pltpu.sync_copy(eflat.at[pl.ds(base, PER)], eloc)
        pltpu.sync_copy(sflat.at[pl.ds(base, PER)], sloc)

        hv = zero16
        evs = []
        for k in range(PER // 16):
            ev = eloc[pl.ds(k * 16, 16)]
            evs.append(ev)
            for e in range(NEXP):
                pc = _lanesum(_eqmask(ev, e), lane)
                hv = hv + _eqmask(lane, e) * pc
        hv_v[...] = hv
        pltpu.sync_copy(hv_v, sh_hist.at[s])
        plsc.subcore_barrier()
        pltpu.sync_copy(sh_hist, histcopy)

        tot = zero16
        prior = zero16
        for w in range(NT):
            row = histcopy[w]
            tot = tot + row
            lt = jnp.minimum(jnp.maximum((zero16 + s) - w, 0), 1)
            prior = prior + lt * row
        off_incl = _prefix_incl(tot, lane)
        start = (off_incl - tot) + prior

        cvec = start
        for k in range(PER // 16):
            ev = evs[k]
            dest = zero16
            for e in range(NEXP):
                mi = _eqmask(ev, e)
                ranks = _prefix_incl(mi, lane)
                le = _eqmask(lane, e)
                ce = _lanesum(le * cvec, lane)
                dest = dest + mi * (ce + ranks - 1)
                pc = _lanesum(mi, lane)
                cvec = cvec + le * pc
            row = k // 4
            colo = (k % 4) * 16
            posloc[row, pl.ds(colo, 16)] = dest
            rbuf[row, pl.ds(colo, 16)] = base + k * 16 + lane

        for j in range(4):
            pltpu.sync_copy(posloc.at[j], pos_hbm.at[pl.ds(base + j * 64, 64)])

        zbuf[...] = zero16
        zbuf_f[...] = zero16.astype(jnp.float32)
        for j in range(PER // 16):
            pltpu.sync_copy(zbuf, sh_rowid.at[pl.ds(base + j * 16, 16)])
            pltpu.sync_copy(zbuf_f, sh_ssort.at[pl.ds(base + j * 16, 16)])
        plsc.subcore_barrier()
        for j in range(4):
            pltpu.sync_copy(rbuf.at[j], sh_rowid.at[posloc.at[j]], add=True)
            pltpu.sync_copy(sloc.at[pl.ds(j * 64, 64)],
                            sh_ssort.at[posloc.at[j]], add=True)
        plsc.subcore_barrier()
        pltpu.sync_copy(sh_rowid.at[pl.ds(base, PER)],
                        rowid_hbm.at[pl.ds(base, PER)])
        pltpu.sync_copy(sh_ssort.at[pl.ds(base, PER)],
                        ssort_hbm.at[pl.ds(base, PER)])


def _sc_sort(eflat, sflat):
    mesh = plsc.VectorSubcoreMesh(core_axis_name="c", subcore_axis_name="s")
    return functools.partial(
        pl.kernel, mesh=mesh,
        out_type=[jax.ShapeDtypeStruct((R,), jnp.int32),
                  jax.ShapeDtypeStruct((R,), jnp.int32),
                  jax.ShapeDtypeStruct((R,), jnp.float32)],
        scratch_types=[
            pltpu.VMEM((PER,), jnp.int32),
            pltpu.VMEM((PER,), jnp.float32),
            pltpu.VMEM((16,), jnp.int32),
            pltpu.VMEM((NT, 16), jnp.int32),
            pltpu.VMEM((4, 64), jnp.int32),
            pltpu.VMEM((4, 64), jnp.int32),
            pltpu.VMEM((16,), jnp.int32),
            pltpu.VMEM((16,), jnp.float32),
            pltpu.VMEM_SHARED((NT, 16), jnp.int32),
            pltpu.VMEM_SHARED((R,), jnp.int32),
            pltpu.VMEM_SHARED((R,), jnp.float32),
        ],
    )(_sc_sort_body)(eflat, sflat)


def _sc_gather_body(rowid_hbm, x_hbm, rx_hbm, idxv, tokv, rows, sem):
    c = lax.axis_index("c")
    s = lax.axis_index("s")
    wid = s * 2 + c
    base = wid * 128
    pltpu.sync_copy(rowid_hbm.at[pl.ds(base, 128)], idxv)
    for k in range(8):
        tokv[pl.ds(k * 16, 16)] = lax.shift_right_logical(
            idxv[pl.ds(k * 16, 16)], 1)
    pltpu.async_copy(x_hbm.at[tokv], rows, sem).wait()
    pltpu.sync_copy(rows, rx_hbm.at[pl.ds(base, 128)])


def _sc_gather(rowid, xf):
    mesh = plsc.VectorSubcoreMesh(core_axis_name="c", subcore_axis_name="s")
    out, = functools.partial(
        pl.kernel, mesh=mesh,
        out_type=[jax.ShapeDtypeStruct((R, D), jnp.float32)],
        scratch_types=[
            pltpu.VMEM((128,), jnp.int32),
            pltpu.VMEM((128,), jnp.int32),
            pltpu.VMEM((128, D), jnp.float32),
            pltpu.SemaphoreType.DMA,
        ],
    )(_sc_gather_body)(rowid, xf)
    return out


def _stair_body(sref, rx_ref, ss_ref, w1_ref, w2_ref, ro_ref):
    g = pl.program_id(0)

    @pl.when(g == 0)
    def _zero():
        ro_ref[...] = jnp.zeros_like(ro_ref)

    @pl.when(sref[2, g] == 1)
    def _compute():
        bg = sref[0, g]
        lo = sref[3, g]
        hi = sref[4, g]
        xb = rx_ref[...].astype(jnp.bfloat16)
        h = jnp.maximum(
            jnp.dot(xb, w1_ref[0].astype(jnp.bfloat16),
                    preferred_element_type=jnp.float32), 0.0)
        cont = jnp.dot(h.astype(jnp.bfloat16), w2_ref[0].astype(jnp.bfloat16),
                       preferred_element_type=jnp.float32)
        riota = jax.lax.broadcasted_iota(jnp.int32, (RB, 1), 0) + bg * RB
        msk = (riota >= lo) & (riota < hi)
        contrib = jnp.where(msk, ss_ref[...] * cont, 0.0)
        ro_ref[pl.ds(bg * RB, RB), :] += contrib


def _staircase(sched, rx, ss2, W1, W2):
    grid_spec = pltpu.PrefetchScalarGridSpec(
        num_scalar_prefetch=1,
        grid=(G,),
        in_specs=[
            pl.BlockSpec((RB, D), lambda g, sref: (sref[0, g], 0)),
            pl.BlockSpec((RB, 1), lambda g, sref: (sref[0, g], 0)),
            pl.BlockSpec((1, D, D), lambda g, sref: (sref[1, g], 0, 0)),
            pl.BlockSpec((1, D, D), lambda g, sref: (sref[1, g], 0, 0)),
        ],
        out_specs=pl.BlockSpec((R, D), lambda g, sref: (0, 0)),
    )
    return pl.pallas_call(
        _stair_body,
        grid_spec=grid_spec,
        out_shape=jax.ShapeDtypeStruct((R, D), jnp.float32),
    )(sched, rx, ss2, W1, W2)


def _sc_combine_body(ro_hbm, pos_hbm, out_hbm, pidx, rows, oloc, sem):
    c = lax.axis_index("c")
    s = lax.axis_index("s")
    wid = s * 2 + c
    base_r = wid * 128
    base_t = wid * 64
    for j in range(2):
        pltpu.sync_copy(pos_hbm.at[pl.ds(base_r + 64 * j, 64)], pidx.at[j])
    for j in range(2):
        pltpu.async_copy(ro_hbm.at[pidx.at[j]], rows, sem).wait()

        def tok_body(i, carry):
            for k in range(D // 16):
                a = rows[2 * i, pl.ds(k * 16, 16)]
                b = rows[2 * i + 1, pl.ds(k * 16, 16)]
                oloc[i, pl.ds(k * 16, 16)] = a + b
            return carry

        lax.fori_loop(0, 32, tok_body, 0)
        pltpu.sync_copy(oloc, out_hbm.at[pl.ds(base_t + 32 * j, 32)])


def _sc_combine(ro, pos):
    mesh = plsc.VectorSubcoreMesh(core_axis_name="c", subcore_axis_name="s")
    out, = functools.partial(
        pl.kernel, mesh=mesh,
        out_type=[jax.ShapeDtypeStruct((T, D), jnp.float32)],
        scratch_types=[
            pltpu.VMEM((2, 64), jnp.int32),
            pltpu.VMEM((64, D), jnp.float32),
            pltpu.VMEM((32, D), jnp.float32),
            pltpu.SemaphoreType.DMA,
        ],
    )(_sc_combine_body)(ro, pos)
    return out


def _schedule(cnt8):
    off = jnp.concatenate([jnp.zeros((1,), jnp.int32),
                           jnp.cumsum(cnt8, dtype=jnp.int32)])
    ee = jnp.repeat(jnp.arange(NEXP, dtype=jnp.int32), NB)
    bb = jnp.tile(jnp.arange(NB, dtype=jnp.int32), NEXP)
    lo = jnp.maximum(off[ee], bb * RB)
    hi = jnp.minimum(off[ee + 1], (bb + 1) * RB)
    valid = (lo < hi).astype(jnp.int32)
    gpos = jnp.cumsum(valid) - 1
    tgt = jnp.where(valid == 1, gpos, G)
    sched = jnp.zeros((5, G + 1), jnp.int32)
    sched = sched.at[0, tgt].set(bb)
    sched = sched.at[1, tgt].set(ee)
    sched = sched.at[2, tgt].set(valid)
    sched = sched.at[3, tgt].set(lo)
    sched = sched.at[4, tgt].set(hi)
    return sched[:, :G]


@functools.partial(jax.jit, static_argnames=())
def kernel(x, Wg, W1, W2):
    B, S, _ = x.shape
    xf = x.reshape(T, D)

    idx2, sc2, lb, rz, cnt = _router(xf, Wg)
    eflat = idx2.reshape(R)
    sflat = sc2.reshape(R)
    cnt8 = cnt.reshape(NEXP)

    sched = _schedule(cnt8)
    pos, rowid, ssort = _sc_sort(eflat, sflat)
    rx = _sc_gather(rowid, xf)
    ro = _staircase(sched, rx, ssort.reshape(R, 1), W1, W2)
    out = _sc_combine(ro, pos)

    return (out.reshape(B, S, D), lb.reshape(()), rz.reshape(()),
            cnt8)


# speedup vs baseline: 2.4710x; 2.4710x over previous
import functools

import jax
import jax.numpy as jnp
from jax.experimental import pallas as pl
from jax.experimental.pallas import tpu as pltpu

TOPK = 2
NEXP = 8
NSLOT = 4


def _moe_body(x_hbm, wg_ref, w1_hbm, w2_hbm,
              out_ref, lb_ref, rz_ref, cnt_ref,
              xv, w1v, w2v, w_scr, semx, sem1, sem2):
    pltpu.make_async_copy(x_hbm, xv, semx).start()
    for s in range(NSLOT):
        pltpu.make_async_copy(w1_hbm.at[s], w1v.at[s], sem1.at[s]).start()
        pltpu.make_async_copy(w2_hbm.at[s], w2v.at[s], sem2.at[s]).start()

    pltpu.make_async_copy(x_hbm, xv, semx).wait()
    xf = xv[...]
    logits = jnp.dot(xf, wg_ref[...], preferred_element_type=jnp.float32)
    m = jnp.max(logits, axis=-1, keepdims=True)
    ex = jnp.exp(logits - m)
    ssum = jnp.sum(ex, axis=-1, keepdims=True)
    scores = ex / ssum
    rz = jnp.log(ssum) + m
    rz_ref[0, 0] = jnp.mean(rz * rz)

    col = jax.lax.broadcasted_iota(jnp.int32, scores.shape, 1)
    m1 = jnp.max(scores, axis=-1, keepdims=True)
    idx1 = jnp.min(jnp.where(scores == m1, col, NEXP), axis=-1, keepdims=True)
    sel1 = col == idx1
    s_masked = jnp.where(sel1, -jnp.inf, scores)
    m2 = jnp.max(s_masked, axis=-1, keepdims=True)
    idx2 = jnp.min(jnp.where(s_masked == m2, col, NEXP), axis=-1,
                   keepdims=True)
    sel2 = col == idx2
    picked = sel1 | sel2
    w = jnp.where(picked, scores, 0.0)
    w_scr[...] = w

    counts = jnp.sum(picked.astype(jnp.int32), axis=0)
    cnt_ref[...] = counts[None, :]
    seg_sum = jnp.sum(w, axis=0)
    total = jnp.float32(w.shape[0] * TOPK)
    dist = counts.astype(jnp.float32) / total
    avg = seg_sum / jnp.maximum(counts.astype(jnp.float32), 1.0)
    lb_ref[0, 0] = jnp.sum(dist * avg) * NEXP

    xb = xf.astype(jnp.bfloat16)
    wall = w_scr[...]
    ecol = jax.lax.broadcasted_iota(jnp.int32, wall.shape, 1)

    for e in range(NEXP):
        slot = e % NSLOT
        pltpu.make_async_copy(w1_hbm.at[e], w1v.at[slot], sem1.at[slot]).wait()
        pltpu.make_async_copy(w2_hbm.at[e], w2v.at[slot], sem2.at[slot]).wait()

        we = jnp.sum(jnp.where(ecol == e, wall, 0.0), axis=1, keepdims=True)
        h = jnp.maximum(
            jnp.dot(xb, w1v[slot].astype(jnp.bfloat16),
                    preferred_element_type=jnp.float32), 0.0)
        contrib = jnp.dot(h.astype(jnp.bfloat16),
                          w2v[slot].astype(jnp.bfloat16),
                          preferred_element_type=jnp.float32)

        if e == 0:
            out_ref[...] = we * contrib
        else:
            out_ref[...] += we * contrib

        nxt = e + NSLOT
        if nxt < NEXP:
            pltpu.make_async_copy(w1_hbm.at[nxt], w1v.at[slot],
                                  sem1.at[slot]).start()
            pltpu.make_async_copy(w2_hbm.at[nxt], w2v.at[slot],
                                  sem2.at[slot]).start()


@functools.partial(jax.jit, static_argnames=())
def kernel(x, Wg, W1, W2):
    B, S, D = x.shape
    E = W1.shape[0]
    F = W1.shape[2]
    T = B * S
    xf = x.reshape(T, D)

    out, lb, rz, cnt = pl.pallas_call(
        _moe_body,
        in_specs=[
            pl.BlockSpec(memory_space=pl.ANY),
            pl.BlockSpec((D, E), lambda: (0, 0)),
            pl.BlockSpec(memory_space=pl.ANY),
            pl.BlockSpec(memory_space=pl.ANY),
        ],
        out_specs=[
            pl.BlockSpec((T, D), lambda: (0, 0)),
            pl.BlockSpec(memory_space=pltpu.SMEM),
            pl.BlockSpec(memory_space=pltpu.SMEM),
            pl.BlockSpec((1, E), lambda: (0, 0)),
        ],
        out_shape=[
            jax.ShapeDtypeStruct((T, D), jnp.float32),
            jax.ShapeDtypeStruct((1, 1), jnp.float32),
            jax.ShapeDtypeStruct((1, 1), jnp.float32),
            jax.ShapeDtypeStruct((1, E), jnp.int32),
        ],
        scratch_shapes=[
            pltpu.VMEM((T, D), jnp.float32),
            pltpu.VMEM((NSLOT, D, F), jnp.float32),
            pltpu.VMEM((NSLOT, F, D), jnp.float32),
            pltpu.VMEM((T, NEXP), jnp.float32),
            pltpu.SemaphoreType.DMA,
            pltpu.SemaphoreType.DMA((NSLOT,)),
            pltpu.SemaphoreType.DMA((NSLOT,)),
        ],
    )(xf, Wg, W1, W2)

    return (out.reshape(B, S, D), lb.reshape(()), rz.reshape(()),
            cnt.reshape(E))


# speedup vs baseline: 2.7013x; 1.0932x over previous
import functools

import jax
import jax.numpy as jnp
from jax.experimental import pallas as pl
from jax.experimental.pallas import tpu as pltpu

TOPK = 2
NEXP = 8
NSLOT = 4


def _moe_body(x_ref, wg_ref, w1_hbm, w2_hbm,
              out_ref, lb_ref, rz_ref, cnt_ref,
              w1v, w2v, w_scr, sem1, sem2):
    for s in range(NSLOT):
        pltpu.make_async_copy(w1_hbm.at[s], w1v.at[s], sem1.at[s]).start()
        pltpu.make_async_copy(w2_hbm.at[s], w2v.at[s], sem2.at[s]).start()

    xf = x_ref[...]
    logits = jnp.dot(xf, wg_ref[...], preferred_element_type=jnp.float32)
    m = jnp.max(logits, axis=-1, keepdims=True)
    ex = jnp.exp(logits - m)
    ssum = jnp.sum(ex, axis=-1, keepdims=True)
    scores = ex / ssum
    rz = jnp.log(ssum) + m
    rz_ref[0, 0] = jnp.mean(rz * rz)

    col = jax.lax.broadcasted_iota(jnp.int32, scores.shape, 1)
    m1 = jnp.max(scores, axis=-1, keepdims=True)
    idx1 = jnp.min(jnp.where(scores == m1, col, NEXP), axis=-1, keepdims=True)
    sel1 = col == idx1
    s_masked = jnp.where(sel1, -jnp.inf, scores)
    m2 = jnp.max(s_masked, axis=-1, keepdims=True)
    idx2 = jnp.min(jnp.where(s_masked == m2, col, NEXP), axis=-1,
                   keepdims=True)
    sel2 = col == idx2
    picked = sel1 | sel2
    w = jnp.where(picked, scores, 0.0)
    w_scr[...] = w

    counts = jnp.sum(picked.astype(jnp.int32), axis=0)
    cnt_ref[...] = counts[None, :]
    seg_sum = jnp.sum(w, axis=0)
    total = jnp.float32(w.shape[0] * TOPK)
    dist = counts.astype(jnp.float32) / total
    avg = seg_sum / jnp.maximum(counts.astype(jnp.float32), 1.0)
    lb_ref[0, 0] = jnp.sum(dist * avg) * NEXP

    xb = xf.astype(jnp.bfloat16)
    wall = w_scr[...]
    ecol = jax.lax.broadcasted_iota(jnp.int32, wall.shape, 1)

    for e in range(NEXP):
        slot = e % NSLOT
        pltpu.make_async_copy(w1_hbm.at[e], w1v.at[slot], sem1.at[slot]).wait()
        pltpu.make_async_copy(w2_hbm.at[e], w2v.at[slot], sem2.at[slot]).wait()

        we = jnp.sum(jnp.where(ecol == e, wall, 0.0), axis=1, keepdims=True)
        h = jnp.maximum(
            jnp.dot(xb, w1v[slot].astype(jnp.bfloat16),
                    preferred_element_type=jnp.float32), 0.0)
        contrib = jnp.dot(h.astype(jnp.bfloat16),
                          w2v[slot].astype(jnp.bfloat16),
                          preferred_element_type=jnp.float32)

        if e == 0:
            out_ref[...] = we * contrib
        else:
            out_ref[...] += we * contrib

        nxt = e + NSLOT
        if nxt < NEXP:
            pltpu.make_async_copy(w1_hbm.at[nxt], w1v.at[slot],
                                  sem1.at[slot]).start()
            pltpu.make_async_copy(w2_hbm.at[nxt], w2v.at[slot],
                                  sem2.at[slot]).start()


@functools.partial(jax.jit, static_argnames=())
def kernel(x, Wg, W1, W2):
    B, S, D = x.shape
    E = W1.shape[0]
    F = W1.shape[2]
    T = B * S
    xf = x.reshape(T, D)

    out, lb, rz, cnt = pl.pallas_call(
        _moe_body,
        in_specs=[
            pl.BlockSpec((T, D), lambda: (0, 0)),
            pl.BlockSpec((D, E), lambda: (0, 0)),
            pl.BlockSpec(memory_space=pl.ANY),
            pl.BlockSpec(memory_space=pl.ANY),
        ],
        out_specs=[
            pl.BlockSpec((T, D), lambda: (0, 0)),
            pl.BlockSpec(memory_space=pltpu.SMEM),
            pl.BlockSpec(memory_space=pltpu.SMEM),
            pl.BlockSpec((1, E), lambda: (0, 0)),
        ],
        out_shape=[
            jax.ShapeDtypeStruct((T, D), jnp.float32),
            jax.ShapeDtypeStruct((1, 1), jnp.float32),
            jax.ShapeDtypeStruct((1, 1), jnp.float32),
            jax.ShapeDtypeStruct((1, E), jnp.int32),
        ],
        scratch_shapes=[
            pltpu.VMEM((NSLOT, D, F), jnp.float32),
            pltpu.VMEM((NSLOT, F, D), jnp.float32),
            pltpu.VMEM((T, NEXP), jnp.float32),
            pltpu.SemaphoreType.DMA((NSLOT,)),
            pltpu.SemaphoreType.DMA((NSLOT,)),
        ],
    )(xf, Wg, W1, W2)

    return (out.reshape(B, S, D), lb.reshape(()), rz.reshape(()),
            cnt.reshape(E))
